# baseline (speedup 1.0000x reference)
import functools

import numpy as np
import jax
from jax import lax
import jax.numpy as jnp
from jax.experimental import pallas as pl
from jax.experimental.pallas import tpu as pltpu, tpu_sc as plsc

IN_DIM = 128
HIDDEN_DIM = 64
OUT_DIM = 128
HEADS_SPACE = 2
K = 9
B = 64
H = 28
W = 28
P = H * W


def _build_static_graph():
    ii, jj = np.meshgrid(np.arange(H), np.arange(W), indexing='ij')
    coords = np.stack([ii.ravel(), jj.ravel()], axis=-1).astype(np.float32)
    coords = coords / coords.max()
    d2 = ((coords[:, None, :] - coords[None, :, :]) ** 2).sum(-1)
    np.fill_diagonal(d2, np.inf)
    nbr = np.argsort(d2, axis=1)[:, :K]
    offs = nbr - np.arange(P)[:, None]
    uniq = np.unique(offs)
    nslot = -((len(uniq) + 1) // -8) * 8
    maskadd = np.full((nslot, P), -1e30, np.float32)
    for j, d in enumerate(uniq):
        maskadd[j] = np.where((offs == d).any(axis=1), 0.0, -1e30)
    maskadd[len(uniq)] = 0.0
    assert (maskadd[:len(uniq)] == 0.0).sum() == P * K
    return [int(d) for d in uniq], maskadd, nslot


_OFFSETS, _MASKADD_NP, _NSLOT = _build_static_graph()
_NOFF = len(_OFFSETS)
_PAD = max(abs(d) for d in _OFFSETS)


def _build_expand():
    nsl = _NOFF + 1
    e2 = np.zeros((2 * _NSLOT, nsl * 128), np.float32)
    for j in range(nsl):
        e2[j, j * 128:j * 128 + HIDDEN_DIM] = 1.0
        e2[_NSLOT + j, j * 128 + HIDDEN_DIM:(j + 1) * 128] = 1.0
    return e2


_EXPAND_NP = _build_expand()


def _lrelu(v):
    return jnp.where(v > 0, v, 0.2 * v)


_FRONT = 88


_G = 8


def _gat1_body(x_ref, w1_ref, avec_ref, b1_ref, maskadd_ref, expand_ref,
               xs_ref, isum_ref):
    for g in range(_G):
        _gat1_one(g, x_ref, w1_ref, avec_ref, b1_ref, maskadd_ref,
                  expand_ref, xs_ref, isum_ref)


def _gat1_one(g, x_ref, w1_ref, avec_ref, b1_ref, maskadd_ref, expand_ref,
              xs_ref, isum_ref):
    f32 = jnp.float32
    xm = x_ref[g]
    w1 = w1_ref[...]
    xw = jax.lax.dot_general(xm, w1, (((0,), (0,)), ((), ())),
                             preferred_element_type=f32)
    avw = jnp.dot(w1, avec_ref[...], preferred_element_type=f32)
    aT = jax.lax.dot_general(avw, xm, (((0,), (0,)), ((), ())),
                             preferred_element_type=f32)
    a_sT = aT[0:2]
    a_dT = aT[2:4]
    zl = jnp.zeros((2, _FRONT), f32)
    a_sT_pad = jnp.concatenate([zl, a_sT, zl], axis=1)

    rows0, rows1 = [], []
    for d in _OFFSETS:
        es = _lrelu(a_sT_pad[:, _FRONT + d:_FRONT + d + P] + a_dT)
        rows0.append(es[0:1])
        rows1.append(es[1:2])
    es_self = _lrelu(a_sT + a_dT)
    rows0.append(es_self[0:1])
    rows1.append(es_self[1:2])
    zrows = jnp.zeros((_NSLOT - _NOFF - 1, P), f32)
    maskadd = maskadd_ref[...]
    exn_heads = []
    for rows in (rows0, rows1):
        est = jnp.concatenate(rows + [zrows], axis=0) + maskadd
        mrow = jnp.max(est, axis=0, keepdims=True)
        ex = jnp.exp(est - mrow)
        den = jnp.sum(ex, axis=0, keepdims=True)
        exn_heads.append(ex * (1.0 / (den + 1e-16)))
    exc = jnp.concatenate(exn_heads, axis=0).T
    expand = expand_ref[...]

    zp = jnp.zeros((_FRONT, HEADS_SPACE * HIDDEN_DIM), f32)
    xw_pad = jnp.concatenate([zp, xw, zp], axis=0)
    rots = {}
    for d in _OFFSETS:
        r = (_FRONT + d) % 8
        if r not in rots:
            rots[r] = xw_pad[r:r + P + 2 * _FRONT - 8]
    shifts = []
    for d in _OFFSETS:
        s = _FRONT + d
        r = s % 8
        shifts.append(rots[r][s - r:s - r + P])
    shifts.append(xw)
    num = None
    csz = 7
    for c0 in range(0, _NOFF + 1, csz):
        c1 = min(c0 + csz, _NOFF + 1)
        bc = jax.lax.dot_general(
            exc, expand[:, c0 * 128:c1 * 128], (((1,), (0,)), ((), ())),
            preferred_element_type=f32)
        for j in range(c0, c1):
            t = bc[:, (j - c0) * 128:(j - c0 + 1) * 128] * shifts[j]
            num = t if num is None else num + t

    xs = 0.5 * (num + b1_ref[...])
    xs_ref[g] = xs
    isum_ref[g] = jnp.sum(xs, axis=0, keepdims=True)


def _gat2_body(isum_ref, bidc_ref, bidr_ref, w2_ref, as2_ref, ad2_ref,
               b2_ref, fv_ref):
    emb = isum_ref[...] * (2.0 / P)
    xw2 = jnp.dot(emb, w2_ref[...], preferred_element_type=jnp.float32)
    a_s_row = jax.lax.dot_general(as2_ref[...], xw2, (((1,), (1,)), ((), ())),
                                  preferred_element_type=jnp.float32)
    a_d_col = jnp.dot(xw2, ad2_ref[...],
                      preferred_element_type=jnp.float32)
    e = _lrelu(a_s_row + a_d_col)
    mval = bidc_ref[...] == bidr_ref[...]
    em = jnp.where(mval, e, -1e30)
    mrow = jnp.max(em, axis=1, keepdims=True)
    ex = jnp.exp(em - mrow) * mval.astype(jnp.float32)
    den = jnp.sum(ex, axis=1, keepdims=True) + 1e-16
    fv = jnp.dot(ex, xw2, preferred_element_type=jnp.float32) / den
    fv_ref[...] = 0.5 * (fv + b2_ref[...])


def _blend_body(xs_ref, fv_ref, o_ref):
    o_ref[0] = 0.5 * xs_ref[0] + 0.5 * fv_ref[0]


_SC_CHUNK = 112


def _make_sc_blend():
    nc, ns = 2, 16
    nw = nc * ns
    img_per_w = B // nw
    nchunk = P // _SC_CHUNK
    mesh = plsc.VectorSubcoreMesh(core_axis_name="c", subcore_axis_name="s")

    @functools.partial(
        pl.kernel, mesh=mesh,
        out_type=jax.ShapeDtypeStruct((B * P, OUT_DIM), jnp.float32),
        scratch_types=[
            pltpu.VMEM((_SC_CHUNK, OUT_DIM), jnp.float32),
            pltpu.VMEM((OUT_DIM,), jnp.float32),
        ],
    )
    def blend_sc(xs_hbm, fv_hbm, out_hbm, chunk_v, fv_v):
        wid = lax.axis_index("s") * nc + lax.axis_index("c")
        for im in range(img_per_w):
            img = wid * img_per_w + im
            pltpu.sync_copy(fv_hbm.at[img], fv_v)
            for c in range(nchunk):
                row0 = img * P + c * _SC_CHUNK
                pltpu.sync_copy(xs_hbm.at[pl.ds(row0, _SC_CHUNK)], chunk_v)

                def body(r, _):
                    for j in range(OUT_DIM // 16):
                        sl = pl.ds(j * 16, 16)
                        chunk_v[r, sl] = chunk_v[r, sl] + fv_v[sl]
                    return 0

                lax.fori_loop(0, _SC_CHUNK, body, 0)
                pltpu.sync_copy(chunk_v, out_hbm.at[pl.ds(row0, _SC_CHUNK)])

    return blend_sc


_SC_BLEND = _make_sc_blend()


def kernel(x, batch_id_all, batch_id, W1, a_src1, a_dst1, b1,
           W2, a_src2, a_dst2, b2):
    del batch_id
    f32 = jnp.float32
    x2 = x.reshape(B, IN_DIM, P)

    z = jnp.zeros((HIDDEN_DIM,), f32)
    avec = jnp.stack([
        jnp.concatenate([a_src1[0], z]),
        jnp.concatenate([z, a_src1[1]]),
        jnp.concatenate([a_dst1[0], z]),
        jnp.concatenate([z, a_dst1[1]]),
    ], axis=1)

    maskadd = jnp.asarray(_MASKADD_NP)

    xs, isum = pl.pallas_call(
        _gat1_body,
        grid=(B // _G,),
        in_specs=[
            pl.BlockSpec((_G, IN_DIM, P), lambda i: (i, 0, 0)),
            pl.BlockSpec((IN_DIM, HEADS_SPACE * HIDDEN_DIM), lambda i: (0, 0)),
            pl.BlockSpec((IN_DIM, 4), lambda i: (0, 0)),
            pl.BlockSpec((1, HEADS_SPACE * HIDDEN_DIM), lambda i: (0, 0)),
            pl.BlockSpec((_NSLOT, P), lambda i: (0, 0)),
            pl.BlockSpec(_EXPAND_NP.shape, lambda i: (0, 0)),
        ],
        out_specs=[
            pl.BlockSpec((_G, P, HEADS_SPACE * HIDDEN_DIM), lambda i: (i, 0, 0)),
            pl.BlockSpec((_G, 1, HEADS_SPACE * HIDDEN_DIM), lambda i: (i, 0, 0)),
        ],
        out_shape=[
            jax.ShapeDtypeStruct((B, P, HEADS_SPACE * HIDDEN_DIM), f32),
            jax.ShapeDtypeStruct((B, 1, HEADS_SPACE * HIDDEN_DIM), f32),
        ],
    )(x2, W1, avec, b1.reshape(1, -1), maskadd, jnp.asarray(_EXPAND_NP))
    isum = isum.reshape(B, HEADS_SPACE * HIDDEN_DIM)

    fv = pl.pallas_call(
        _gat2_body,
        out_shape=jax.ShapeDtypeStruct((B, OUT_DIM), f32),
    )(isum, batch_id_all.reshape(B, 1), batch_id_all.reshape(1, B),
      W2, a_src2, a_dst2.reshape(-1, 1), b2.reshape(1, -1))

    out = _SC_BLEND(xs.reshape(B * P, OUT_DIM), fv)

    return out.reshape(B, OUT_DIM, H, W)


# speedup vs baseline: 1.0199x; 1.0199x over previous
import functools

import numpy as np
import jax
from jax import lax
import jax.numpy as jnp
from jax.experimental import pallas as pl
from jax.experimental.pallas import tpu as pltpu, tpu_sc as plsc

IN_DIM = 128
HIDDEN_DIM = 64
OUT_DIM = 128
HEADS_SPACE = 2
K = 9
B = 64
H = 28
W = 28
P = H * W


def _build_static_graph():
    ii, jj = np.meshgrid(np.arange(H), np.arange(W), indexing='ij')
    coords = np.stack([ii.ravel(), jj.ravel()], axis=-1).astype(np.float32)
    coords = coords / coords.max()
    d2 = ((coords[:, None, :] - coords[None, :, :]) ** 2).sum(-1)
    np.fill_diagonal(d2, np.inf)
    nbr = np.argsort(d2, axis=1)[:, :K]
    offs = nbr - np.arange(P)[:, None]
    uniq = np.unique(offs)
    nslot = -((len(uniq) + 1) // -8) * 8
    maskadd = np.full((nslot, P), -1e30, np.float32)
    for j, d in enumerate(uniq):
        maskadd[j] = np.where((offs == d).any(axis=1), 0.0, -1e30)
    maskadd[len(uniq)] = 0.0
    assert (maskadd[:len(uniq)] == 0.0).sum() == P * K
    return [int(d) for d in uniq], maskadd, nslot


_OFFSETS, _MASKADD_NP, _NSLOT = _build_static_graph()
_NOFF = len(_OFFSETS)
_PAD = max(abs(d) for d in _OFFSETS)


def _build_expand():
    nsl = _NOFF + 1
    e2 = np.zeros((2 * _NSLOT, nsl * 128), np.float32)
    for j in range(nsl):
        e2[j, j * 128:j * 128 + HIDDEN_DIM] = 1.0
        e2[_NSLOT + j, j * 128 + HIDDEN_DIM:(j + 1) * 128] = 1.0
    return e2


_EXPAND_NP = _build_expand()


def _lrelu(v):
    return jnp.where(v > 0, v, 0.2 * v)


_FRONT = 88


_G = 8


def _gat1_body(x_ref, w1_ref, avec_ref, b1_ref, maskadd_ref, expand_ref,
               xs_ref, isum_ref):
    for g in range(_G):
        _gat1_one(g, x_ref, w1_ref, avec_ref, b1_ref, maskadd_ref,
                  expand_ref, xs_ref, isum_ref)


def _gat1_one(g, x_ref, w1_ref, avec_ref, b1_ref, maskadd_ref, expand_ref,
              xs_ref, isum_ref):
    f32 = jnp.float32
    xm = x_ref[g]
    w1 = w1_ref[...]
    xw = jax.lax.dot_general(xm, w1, (((0,), (0,)), ((), ())),
                             preferred_element_type=f32)
    avw = jnp.dot(w1, avec_ref[...], preferred_element_type=f32)
    aT = jax.lax.dot_general(avw, xm, (((0,), (0,)), ((), ())),
                             preferred_element_type=f32)
    a_sT = aT[0:2]
    a_dT = aT[2:4]
    zl = jnp.zeros((2, _FRONT), f32)
    a_sT_pad = jnp.concatenate([zl, a_sT, zl], axis=1)

    rows0, rows1 = [], []
    for d in _OFFSETS:
        es = _lrelu(a_sT_pad[:, _FRONT + d:_FRONT + d + P] + a_dT)
        rows0.append(es[0:1])
        rows1.append(es[1:2])
    es_self = _lrelu(a_sT + a_dT)
    rows0.append(es_self[0:1])
    rows1.append(es_self[1:2])
    zrows = jnp.zeros((_NSLOT - _NOFF - 1, P), f32)
    maskadd = maskadd_ref[...]
    exn_heads = []
    for rows in (rows0, rows1):
        est = jnp.concatenate(rows + [zrows], axis=0) + maskadd
        mrow = jnp.max(est, axis=0, keepdims=True)
        ex = jnp.exp(est - mrow)
        den = jnp.sum(ex, axis=0, keepdims=True)
        exn_heads.append(ex * (1.0 / (den + 1e-16)))
    exc = jnp.concatenate(exn_heads, axis=0).T
    expand = expand_ref[...]

    zp = jnp.zeros((_FRONT, HEADS_SPACE * HIDDEN_DIM), f32)
    xw_pad = jnp.concatenate([zp, xw, zp], axis=0)
    rots = {}
    for d in _OFFSETS:
        r = (_FRONT + d) % 8
        if r not in rots:
            rots[r] = xw_pad[r:r + P + 2 * _FRONT - 8]
    shifts = []
    for d in _OFFSETS:
        s = _FRONT + d
        r = s % 8
        shifts.append(rots[r][s - r:s - r + P])
    shifts.append(xw)
    num = None
    csz = 7
    for c0 in range(0, _NOFF + 1, csz):
        c1 = min(c0 + csz, _NOFF + 1)
        bc = jax.lax.dot_general(
            exc, expand[:, c0 * 128:c1 * 128], (((1,), (0,)), ((), ())),
            preferred_element_type=f32)
        for j in range(c0, c1):
            t = bc[:, (j - c0) * 128:(j - c0 + 1) * 128] * shifts[j]
            num = t if num is None else num + t

    xs = 0.5 * (num + b1_ref[...])
    xs_ref[g] = xs
    isum_ref[g] = jnp.sum(xs, axis=0, keepdims=True)


def _gat2_body(isum_ref, bidc_ref, bidr_ref, w2_ref, as2_ref, ad2_ref,
               b2_ref, fv_ref):
    emb = isum_ref[...] * (2.0 / P)
    xw2 = jnp.dot(emb, w2_ref[...], preferred_element_type=jnp.float32)
    a_s_row = jax.lax.dot_general(as2_ref[...], xw2, (((1,), (1,)), ((), ())),
                                  preferred_element_type=jnp.float32)
    a_d_col = jnp.dot(xw2, ad2_ref[...],
                      preferred_element_type=jnp.float32)
    e = _lrelu(a_s_row + a_d_col)
    mval = bidc_ref[...] == bidr_ref[...]
    em = jnp.where(mval, e, -1e30)
    mrow = jnp.max(em, axis=1, keepdims=True)
    ex = jnp.exp(em - mrow) * mval.astype(jnp.float32)
    den = jnp.sum(ex, axis=1, keepdims=True) + 1e-16
    fv = jnp.dot(ex, xw2, preferred_element_type=jnp.float32) / den
    fv_ref[...] = 0.5 * (fv + b2_ref[...])


def _blend_body(xs_ref, fv_ref, o_ref):
    o_ref[0] = 0.5 * xs_ref[0] + 0.5 * fv_ref[0]


_SC_CHUNK = 392


def _make_sc_blend():
    nc, ns = 2, 16
    nw = nc * ns
    img_per_w = B // nw
    nchunk = P // _SC_CHUNK
    mesh = plsc.VectorSubcoreMesh(core_axis_name="c", subcore_axis_name="s")

    @functools.partial(
        pl.kernel, mesh=mesh,
        out_type=jax.ShapeDtypeStruct((B * P, OUT_DIM), jnp.float32),
        scratch_types=[
            pltpu.VMEM((_SC_CHUNK, OUT_DIM), jnp.float32),
            pltpu.VMEM((OUT_DIM,), jnp.float32),
        ],
    )
    def blend_sc(xs_hbm, fv_hbm, out_hbm, chunk_v, fv_v):
        wid = lax.axis_index("s") * nc + lax.axis_index("c")
        for im in range(img_per_w):
            img = wid * img_per_w + im
            pltpu.sync_copy(fv_hbm.at[img], fv_v)
            for c in range(nchunk):
                row0 = img * P + c * _SC_CHUNK
                pltpu.sync_copy(xs_hbm.at[pl.ds(row0, _SC_CHUNK)], chunk_v)

                def body(r4, _):
                    for u in range(4):
                        r = r4 * 4 + u
                        for j in range(OUT_DIM // 16):
                            sl = pl.ds(j * 16, 16)
                            chunk_v[r, sl] = chunk_v[r, sl] + fv_v[sl]
                    return 0

                lax.fori_loop(0, _SC_CHUNK // 4, body, 0)
                pltpu.sync_copy(chunk_v, out_hbm.at[pl.ds(row0, _SC_CHUNK)])

    return blend_sc


_SC_BLEND = _make_sc_blend()


def kernel(x, batch_id_all, batch_id, W1, a_src1, a_dst1, b1,
           W2, a_src2, a_dst2, b2):
    del batch_id
    f32 = jnp.float32
    x2 = x.reshape(B, IN_DIM, P)

    z = jnp.zeros((HIDDEN_DIM,), f32)
    avec = jnp.stack([
        jnp.concatenate([a_src1[0], z]),
        jnp.concatenate([z, a_src1[1]]),
        jnp.concatenate([a_dst1[0], z]),
        jnp.concatenate([z, a_dst1[1]]),
    ], axis=1)

    maskadd = jnp.asarray(_MASKADD_NP)

    xs, isum = pl.pallas_call(
        _gat1_body,
        grid=(B // _G,),
        in_specs=[
            pl.BlockSpec((_G, IN_DIM, P), lambda i: (i, 0, 0)),
            pl.BlockSpec((IN_DIM, HEADS_SPACE * HIDDEN_DIM), lambda i: (0, 0)),
            pl.BlockSpec((IN_DIM, 4), lambda i: (0, 0)),
            pl.BlockSpec((1, HEADS_SPACE * HIDDEN_DIM), lambda i: (0, 0)),
            pl.BlockSpec((_NSLOT, P), lambda i: (0, 0)),
            pl.BlockSpec(_EXPAND_NP.shape, lambda i: (0, 0)),
        ],
        out_specs=[
            pl.BlockSpec((_G, P, HEADS_SPACE * HIDDEN_DIM), lambda i: (i, 0, 0)),
            pl.BlockSpec((_G, 1, HEADS_SPACE * HIDDEN_DIM), lambda i: (i, 0, 0)),
        ],
        out_shape=[
            jax.ShapeDtypeStruct((B, P, HEADS_SPACE * HIDDEN_DIM), f32),
            jax.ShapeDtypeStruct((B, 1, HEADS_SPACE * HIDDEN_DIM), f32),
        ],
    )(x2, W1, avec, b1.reshape(1, -1), maskadd, jnp.asarray(_EXPAND_NP))
    isum = isum.reshape(B, HEADS_SPACE * HIDDEN_DIM)

    fv = pl.pallas_call(
        _gat2_body,
        out_shape=jax.ShapeDtypeStruct((B, OUT_DIM), f32),
    )(isum, batch_id_all.reshape(B, 1), batch_id_all.reshape(1, B),
      W2, a_src2, a_dst2.reshape(-1, 1), b2.reshape(1, -1))

    out = _SC_BLEND(xs.reshape(B * P, OUT_DIM), fv)

    return out.reshape(B, OUT_DIM, H, W)


# speedup vs baseline: 1.0757x; 1.0548x over previous
import functools

import numpy as np
import jax
from jax import lax
import jax.numpy as jnp
from jax.experimental import pallas as pl
from jax.experimental.pallas import tpu as pltpu, tpu_sc as plsc

IN_DIM = 128
HIDDEN_DIM = 64
OUT_DIM = 128
HEADS_SPACE = 2
K = 9
B = 64
H = 28
W = 28
P = H * W


def _build_static_graph():
    ii, jj = np.meshgrid(np.arange(H), np.arange(W), indexing='ij')
    coords = np.stack([ii.ravel(), jj.ravel()], axis=-1).astype(np.float32)
    coords = coords / coords.max()
    d2 = ((coords[:, None, :] - coords[None, :, :]) ** 2).sum(-1)
    np.fill_diagonal(d2, np.inf)
    nbr = np.argsort(d2, axis=1)[:, :K]
    offs = nbr - np.arange(P)[:, None]
    uniq = np.unique(offs)
    nslot = -((len(uniq) + 1) // -8) * 8
    maskadd = np.full((nslot, P), -1e30, np.float32)
    for j, d in enumerate(uniq):
        maskadd[j] = np.where((offs == d).any(axis=1), 0.0, -1e30)
    maskadd[len(uniq)] = 0.0
    assert (maskadd[:len(uniq)] == 0.0).sum() == P * K
    return [int(d) for d in uniq], maskadd, nslot


_OFFSETS, _MASKADD_NP, _NSLOT = _build_static_graph()
_NOFF = len(_OFFSETS)
_PAD = max(abs(d) for d in _OFFSETS)


def _build_expand():
    nsl = _NOFF + 1
    e2 = np.zeros((2 * _NSLOT, nsl * 128), np.float32)
    for j in range(nsl):
        e2[j, j * 128:j * 128 + HIDDEN_DIM] = 1.0
        e2[_NSLOT + j, j * 128 + HIDDEN_DIM:(j + 1) * 128] = 1.0
    return e2


_EXPAND_NP = _build_expand()


def _lrelu(v):
    return jnp.where(v > 0, v, 0.2 * v)


_FRONT = 88


_G = 8


def _gat1_body(x_ref, w1_ref, avec_ref, b1_ref, maskadd_ref, expand_ref,
               xs_ref, isum_ref):
    for g in range(_G):
        _gat1_one(g, x_ref, w1_ref, avec_ref, b1_ref, maskadd_ref,
                  expand_ref, xs_ref, isum_ref)


def _gat1_one(g, x_ref, w1_ref, avec_ref, b1_ref, maskadd_ref, expand_ref,
              xs_ref, isum_ref):
    f32 = jnp.float32
    xm = x_ref[g]
    w1 = w1_ref[...]
    xw = jax.lax.dot_general(xm, w1, (((0,), (0,)), ((), ())),
                             preferred_element_type=f32)
    avw = jnp.dot(w1, avec_ref[...], preferred_element_type=f32)
    aT = jax.lax.dot_general(avw, xm, (((0,), (0,)), ((), ())),
                             preferred_element_type=f32)
    a_sT = aT[0:2]
    a_dT = aT[2:4]
    zl = jnp.zeros((2, _FRONT), f32)
    a_sT_pad = jnp.concatenate([zl, a_sT, zl], axis=1)

    rows0, rows1 = [], []
    for d in _OFFSETS:
        es = _lrelu(a_sT_pad[:, _FRONT + d:_FRONT + d + P] + a_dT)
        rows0.append(es[0:1])
        rows1.append(es[1:2])
    es_self = _lrelu(a_sT + a_dT)
    rows0.append(es_self[0:1])
    rows1.append(es_self[1:2])
    zrows = jnp.zeros((_NSLOT - _NOFF - 1, P), f32)
    maskadd = maskadd_ref[...]
    exn_heads = []
    for rows in (rows0, rows1):
        est = jnp.concatenate(rows + [zrows], axis=0) + maskadd
        mrow = jnp.max(est, axis=0, keepdims=True)
        ex = jnp.exp(est - mrow)
        den = jnp.sum(ex, axis=0, keepdims=True)
        exn_heads.append(ex * (1.0 / (den + 1e-16)))
    exc = jnp.concatenate(exn_heads, axis=0).T
    expand = expand_ref[...]

    zp = jnp.zeros((_FRONT, HEADS_SPACE * HIDDEN_DIM), f32)
    xw_pad = jnp.concatenate([zp, xw, zp], axis=0)
    rots = {}
    for d in _OFFSETS:
        r = (_FRONT + d) % 8
        if r not in rots:
            rots[r] = xw_pad[r:r + P + 2 * _FRONT - 8]
    shifts = []
    for d in _OFFSETS:
        s = _FRONT + d
        r = s % 8
        shifts.append(rots[r][s - r:s - r + P])
    shifts.append(xw)
    num = None
    csz = 7
    for c0 in range(0, _NOFF + 1, csz):
        c1 = min(c0 + csz, _NOFF + 1)
        bc = jax.lax.dot_general(
            exc, expand[:, c0 * 128:c1 * 128], (((1,), (0,)), ((), ())),
            preferred_element_type=f32)
        for j in range(c0, c1):
            t = bc[:, (j - c0) * 128:(j - c0 + 1) * 128] * shifts[j]
            num = t if num is None else num + t

    xs = 0.5 * (num + b1_ref[...])
    xs_ref[g] = xs
    isum_ref[g] = jnp.sum(xs, axis=0, keepdims=True)


def _gat2_body(isum_ref, bidc_ref, bidr_ref, w2_ref, as2_ref, ad2_ref,
               b2_ref, fv_ref):
    emb = isum_ref[...] * (2.0 / P)
    xw2 = jnp.dot(emb, w2_ref[...], preferred_element_type=jnp.float32)
    a_s_row = jax.lax.dot_general(as2_ref[...], xw2, (((1,), (1,)), ((), ())),
                                  preferred_element_type=jnp.float32)
    a_d_col = jnp.dot(xw2, ad2_ref[...],
                      preferred_element_type=jnp.float32)
    e = _lrelu(a_s_row + a_d_col)
    mval = bidc_ref[...] == bidr_ref[...]
    em = jnp.where(mval, e, -1e30)
    mrow = jnp.max(em, axis=1, keepdims=True)
    ex = jnp.exp(em - mrow) * mval.astype(jnp.float32)
    den = jnp.sum(ex, axis=1, keepdims=True) + 1e-16
    fv = jnp.dot(ex, xw2, preferred_element_type=jnp.float32) / den
    fv_ref[...] = 0.5 * (fv + b2_ref[...])


def _blend_body(xs_ref, fv_ref, o_ref):
    o_ref[0] = xs_ref[0] + fv_ref[0]


_SC_CHUNK = 392


def _make_sc_blend():
    nc, ns = 2, 16
    nw = nc * ns
    img_per_w = 1
    nchunk = P // _SC_CHUNK
    mesh = plsc.VectorSubcoreMesh(core_axis_name="c", subcore_axis_name="s")

    @functools.partial(
        pl.kernel, mesh=mesh,
        out_type=jax.ShapeDtypeStruct((B * P // 2, OUT_DIM), jnp.float32),
        scratch_types=[
            pltpu.VMEM((_SC_CHUNK, OUT_DIM), jnp.float32),
            pltpu.VMEM((OUT_DIM,), jnp.float32),
        ],
    )
    def blend_sc(xs_hbm, fv_hbm, out_hbm, chunk_v, fv_v):
        wid = lax.axis_index("s") * nc + lax.axis_index("c")
        for im in range(img_per_w):
            img = wid * img_per_w + im
            pltpu.sync_copy(fv_hbm.at[img], fv_v)
            for c in range(nchunk):
                row0 = img * P + c * _SC_CHUNK
                pltpu.sync_copy(xs_hbm.at[pl.ds(row0, _SC_CHUNK)], chunk_v)

                def body(r4, _):
                    for u in range(4):
                        r = r4 * 4 + u
                        for j in range(OUT_DIM // 16):
                            sl = pl.ds(j * 16, 16)
                            chunk_v[r, sl] = chunk_v[r, sl] + fv_v[sl]
                    return 0

                lax.fori_loop(0, _SC_CHUNK // 4, body, 0)
                pltpu.sync_copy(chunk_v, out_hbm.at[pl.ds(row0, _SC_CHUNK)])

    return blend_sc


_SC_BLEND = _make_sc_blend()


def kernel(x, batch_id_all, batch_id, W1, a_src1, a_dst1, b1,
           W2, a_src2, a_dst2, b2):
    del batch_id
    f32 = jnp.float32
    x2 = x.reshape(B, IN_DIM, P)

    z = jnp.zeros((HIDDEN_DIM,), f32)
    avec = jnp.stack([
        jnp.concatenate([a_src1[0], z]),
        jnp.concatenate([z, a_src1[1]]),
        jnp.concatenate([a_dst1[0], z]),
        jnp.concatenate([z, a_dst1[1]]),
    ], axis=1)

    maskadd = jnp.asarray(_MASKADD_NP)

    xs, isum = pl.pallas_call(
        _gat1_body,
        grid=(B // _G,),
        in_specs=[
            pl.BlockSpec((_G, IN_DIM, P), lambda i: (i, 0, 0)),
            pl.BlockSpec((IN_DIM, HEADS_SPACE * HIDDEN_DIM), lambda i: (0, 0)),
            pl.BlockSpec((IN_DIM, 4), lambda i: (0, 0)),
            pl.BlockSpec((1, HEADS_SPACE * HIDDEN_DIM), lambda i: (0, 0)),
            pl.BlockSpec((_NSLOT, P), lambda i: (0, 0)),
            pl.BlockSpec(_EXPAND_NP.shape, lambda i: (0, 0)),
        ],
        out_specs=[
            pl.BlockSpec((_G, P, HEADS_SPACE * HIDDEN_DIM), lambda i: (i, 0, 0)),
            pl.BlockSpec((_G, 1, HEADS_SPACE * HIDDEN_DIM), lambda i: (i, 0, 0)),
        ],
        out_shape=[
            jax.ShapeDtypeStruct((B, P, HEADS_SPACE * HIDDEN_DIM), f32),
            jax.ShapeDtypeStruct((B, 1, HEADS_SPACE * HIDDEN_DIM), f32),
        ],
    )(x2, W1, avec, b1.reshape(1, -1), maskadd, jnp.asarray(_EXPAND_NP))
    isum = isum.reshape(B, HEADS_SPACE * HIDDEN_DIM)

    fv = pl.pallas_call(
        _gat2_body,
        out_shape=jax.ShapeDtypeStruct((B, OUT_DIM), f32),
    )(isum, batch_id_all.reshape(B, 1), batch_id_all.reshape(1, B),
      W2, a_src2, a_dst2.reshape(-1, 1), b2.reshape(1, -1))

    hb = B // 2
    out_tc = pl.pallas_call(
        _blend_body,
        grid=(hb,),
        in_specs=[
            pl.BlockSpec((1, P, OUT_DIM), lambda i: (i, 0, 0)),
            pl.BlockSpec((1, 1, OUT_DIM), lambda i: (i, 0, 0)),
        ],
        out_specs=pl.BlockSpec((1, P, OUT_DIM), lambda i: (i, 0, 0)),
        out_shape=jax.ShapeDtypeStruct((hb, P, OUT_DIM), f32),
    )(xs[:hb], fv[:hb].reshape(hb, 1, OUT_DIM))
    out_sc = _SC_BLEND(xs[hb:].reshape(hb * P, OUT_DIM), fv[hb:])

    out = jnp.concatenate([out_tc.reshape(hb, P, OUT_DIM),
                           out_sc.reshape(hb, P, OUT_DIM)], axis=0)
    return out.reshape(B, OUT_DIM, H, W)
